# baseline (speedup 1.0000x reference)
import functools

import jax
import jax.numpy as jnp
from jax import lax
from jax.experimental import pallas as pl
from jax.experimental.pallas import tpu as pltpu
from jax.experimental.pallas import tpu_sc as plsc

VOCAB = 1000
HIDDEN = 128
BATCH = 1024
SEQ = 50
TPAD = 1024
NBLK = VOCAB // 8

NC = 2
NS = 16
NW = NC * NS

VBLK = 4
VROWS = 8 * VBLK
LAST_START = NBLK - VBLK
NBT = BATCH // 128
NGRP = BATCH // 16


def _table_body(w_ref, e_ref, b_ref, out_ref):
    m = (
        lax.dot_general(
            w_ref[...],
            e_ref[...],
            (((1,), (1,)), ((), ())),
            preferred_element_type=jnp.float32,
            precision=lax.Precision.HIGHEST,
        )
        + b_ref[...]
    )
    for j in range(8):
        out_ref[:, j, :] = m[:, 128 * j : 128 * (j + 1)]


def _compute_table(embed, W, b):
    embed_pad = jnp.pad(embed, ((0, TPAD - VOCAB), (0, 0)))
    out3 = pl.pallas_call(
        _table_body,
        out_shape=jax.ShapeDtypeStruct((VOCAB, 8, 128), jnp.float32),
    )(W, embed_pad, b.reshape(VOCAB, 1))
    return out3.reshape(VOCAB, TPAD)


_MESH = plsc.VectorSubcoreMesh(
    core_axis_name="c", subcore_axis_name="s", num_cores=NC, num_subcores=NS
)


@functools.partial(
    pl.kernel,
    out_type=jax.ShapeDtypeStruct((SEQ, NBLK, 8, 8, 128), jnp.float32),
    mesh=_MESH,
    scratch_types=[
        pltpu.VMEM((VROWS, TPAD), jnp.float32),
        pltpu.VMEM((BATCH,), jnp.int32),
        pltpu.VMEM((BATCH,), jnp.int32),
        pltpu.VMEM((VBLK, 8, 8, 128), jnp.float32),
        pltpu.VMEM((VBLK, 8, 8, 128), jnp.float32),
        pltpu.SemaphoreType.DMA,
        pltpu.SemaphoreType.DMA,
        pltpu.SemaphoreType.DMA,
        pltpu.SemaphoreType.DMA,
    ],
    compiler_params=pltpu.CompilerParams(
        use_tc_tiling_on_sc=False, needs_layout_passes=False
    ),
)
def _lookup(
    table_hbm, ids_hbm, out_hbm,
    tbl, idx0, idx1, pan0, pan1, semw0, semw1, semi0, semi1,
):
    wid = lax.axis_index("s") * NC + lax.axis_index("c")
    bs = jnp.minimum(VBLK * wid, LAST_START)

    pltpu.sync_copy(table_hbm.at[pl.ds(8 * bs, VROWS)], tbl)

    idxs = (idx0, idx1)
    pans = (pan0, pan1)
    semw = (semw0, semw1)
    semi = (semi0, semi1)

    def start_idx(s, p):
        pltpu.async_copy(ids_hbm.at[s], idxs[p], semi[p])

    def wait_idx(p):
        pltpu.make_async_copy(ids_hbm.at[0], idxs[p], semi[p]).wait()

    def compute(p):
        @plsc.parallel_loop(0, NGRP, 1, unroll=2)
        def _(g):
            bt = g // 8
            gr = g - 8 * bt
            tv = idxs[p][pl.ds(g * 16, 16)]
            vals = [
                plsc.load_gather(tbl.at[8 * vb + vs], [tv])
                for vb in range(VBLK)
                for vs in range(8)
            ]
            for vb in range(VBLK):
                for vs in range(8):
                    pans[p][vb, bt, vs, pl.ds(gr * 16, 16)] = vals[8 * vb + vs]

    def start_write(s, p):
        pltpu.async_copy(pans[p], out_hbm.at[s, pl.ds(bs, VBLK)], semw[p])

    def wait_write(p):
        pltpu.make_async_copy(pans[p], out_hbm.at[0, pl.ds(bs, VBLK)], semw[p]).wait()

    start_idx(0, 0)
    start_idx(1, 1)
    wait_idx(0)
    compute(0)
    start_write(0, 0)
    wait_idx(1)
    compute(1)
    start_write(1, 1)

    def pair(t, carry):
        s0 = 2 * t
        start_idx(s0, 0)
        start_idx(s0 + 1, 1)
        wait_write(0)
        wait_idx(0)
        compute(0)
        start_write(s0, 0)
        wait_write(1)
        wait_idx(1)
        compute(1)
        start_write(s0 + 1, 1)
        return carry

    lax.fori_loop(1, SEQ // 2, pair, 0)
    wait_write(0)
    wait_write(1)


def kernel(input_ids, embed, W, b):
    table = _compute_table(embed, W, b)
    ids_t = input_ids.T.astype(jnp.int32)
    out5 = _lookup(table, ids_t)
    x = out5.transpose(2, 4, 0, 1, 3)
    return x.reshape(BATCH, SEQ, VOCAB)


# speedup vs baseline: 1.1222x; 1.1222x over previous
import functools

import jax
import jax.numpy as jnp
from jax import lax
from jax.experimental import pallas as pl
from jax.experimental.pallas import tpu as pltpu
from jax.experimental.pallas import tpu_sc as plsc

VOCAB = 1000
HIDDEN = 128
BATCH = 1024
SEQ = 50
TPAD = 1024
NBLK = VOCAB // 8

NC = 2
NS = 16
NW = NC * NS

VBLK = 4
VROWS = 8 * VBLK
LAST_START = NBLK - VBLK
NBT = BATCH // 128
NGRP = BATCH // 16


def _table_body(w_ref, e_ref, b_ref, out_ref):
    m = (
        lax.dot_general(
            w_ref[...],
            e_ref[...],
            (((1,), (1,)), ((), ())),
            preferred_element_type=jnp.float32,
            precision=lax.Precision.HIGHEST,
        )
        + b_ref[...]
    )
    for j in range(8):
        out_ref[:, j, :] = m[:, 128 * j : 128 * (j + 1)]


def _compute_table(embed, W, b):
    embed_pad = jnp.pad(embed, ((0, TPAD - VOCAB), (0, 0)))
    out3 = pl.pallas_call(
        _table_body,
        out_shape=jax.ShapeDtypeStruct((VOCAB, 8, 128), jnp.float32),
    )(W, embed_pad, b.reshape(VOCAB, 1))
    return out3.reshape(VOCAB, TPAD)


_MESH = plsc.VectorSubcoreMesh(
    core_axis_name="c", subcore_axis_name="s", num_cores=NC, num_subcores=NS
)


@functools.partial(
    pl.kernel,
    out_type=jax.ShapeDtypeStruct((SEQ, NBLK, 8, 8, 128), jnp.float32),
    mesh=_MESH,
    scratch_types=[
        pltpu.VMEM((VROWS, TPAD), jnp.float32),
        pltpu.VMEM((BATCH,), jnp.int32),
        pltpu.VMEM((BATCH,), jnp.int32),
        pltpu.VMEM((VBLK, 8, 8, 128), jnp.float32),
        pltpu.VMEM((VBLK, 8, 8, 128), jnp.float32),
        pltpu.SemaphoreType.DMA,
        pltpu.SemaphoreType.DMA,
        pltpu.SemaphoreType.DMA,
        pltpu.SemaphoreType.DMA,
    ],
    compiler_params=pltpu.CompilerParams(
        use_tc_tiling_on_sc=False, needs_layout_passes=False
    ),
)
def _lookup(
    table_hbm, ids_hbm, out_hbm,
    tbl, idx0, idx1, pan0, pan1, semw0, semw1, semi0, semi1,
):
    wid = lax.axis_index("s") * NC + lax.axis_index("c")
    bs = jnp.minimum(VBLK * wid, LAST_START)

    pltpu.sync_copy(table_hbm.at[pl.ds(8 * bs, VROWS)], tbl)

    idxs = (idx0, idx1)
    pans = (pan0, pan1)
    semw = (semw0, semw1)
    semi = (semi0, semi1)

    def start_idx(s, p):
        pltpu.async_copy(ids_hbm.at[s], idxs[p], semi[p])

    def wait_idx(p):
        pltpu.make_async_copy(ids_hbm.at[0], idxs[p], semi[p]).wait()

    def compute(p):
        @plsc.parallel_loop(0, NGRP, 1, unroll=8)
        def _(g):
            bt = g // 8
            gr = g - 8 * bt
            tv = idxs[p][pl.ds(g * 16, 16)]
            for vb in range(VBLK):
                vals = [
                    plsc.load_gather(tbl.at[8 * vb + vs], [tv])
                    for vs in range(8)
                ]
                for vs in range(8):
                    pans[p][vb, bt, vs, pl.ds(gr * 16, 16)] = vals[vs]

    def start_write(s, p):
        pltpu.async_copy(pans[p], out_hbm.at[s, pl.ds(bs, VBLK)], semw[p])

    def wait_write(p):
        pltpu.make_async_copy(pans[p], out_hbm.at[0, pl.ds(bs, VBLK)], semw[p]).wait()

    start_idx(0, 0)
    start_idx(1, 1)
    wait_idx(0)
    compute(0)
    start_write(0, 0)
    wait_idx(1)
    compute(1)
    start_write(1, 1)

    def pair(t, carry):
        s0 = 2 * t
        start_idx(s0, 0)
        start_idx(s0 + 1, 1)
        wait_write(0)
        wait_idx(0)
        compute(0)
        start_write(s0, 0)
        wait_write(1)
        wait_idx(1)
        compute(1)
        start_write(s0 + 1, 1)
        return carry

    lax.fori_loop(1, SEQ // 2, pair, 0)
    wait_write(0)
    wait_write(1)


def kernel(input_ids, embed, W, b):
    table = _compute_table(embed, W, b)
    ids_t = input_ids.T.astype(jnp.int32)
    out5 = _lookup(table, ids_t)
    x = out5.transpose(2, 4, 0, 1, 3)
    return x.reshape(BATCH, SEQ, VOCAB)


# speedup vs baseline: 1.3397x; 1.1938x over previous
import functools

import jax
import jax.numpy as jnp
from jax import lax
from jax.experimental import pallas as pl
from jax.experimental.pallas import tpu as pltpu
from jax.experimental.pallas import tpu_sc as plsc

VOCAB = 1000
HIDDEN = 128
BATCH = 1024
SEQ = 50
TPAD = 1024
NBLK = VOCAB // 8

NC = 2
NS = 16
NW = NC * NS

VBLK = 4
VROWS = 8 * VBLK
LAST_START = NBLK - VBLK
NBT = BATCH // 128
NGRP = BATCH // 16


def _table_body(w_ref, e_ref, b_ref, out_ref):
    m = (
        lax.dot_general(
            w_ref[...],
            e_ref[...],
            (((1,), (1,)), ((), ())),
            preferred_element_type=jnp.float32,
            precision=lax.Precision.HIGHEST,
        )
        + b_ref[...]
    )
    for j in range(8):
        out_ref[:, j, :] = m[:, 128 * j : 128 * (j + 1)]


def _compute_table(embed, W, b):
    embed_pad = jnp.pad(embed, ((0, TPAD - VOCAB), (0, 0)))
    out3 = pl.pallas_call(
        _table_body,
        out_shape=jax.ShapeDtypeStruct((VOCAB, 8, 128), jnp.float32),
    )(W, embed_pad, b.reshape(VOCAB, 1))
    return out3.reshape(VOCAB, TPAD)


_MESH = plsc.VectorSubcoreMesh(
    core_axis_name="c", subcore_axis_name="s", num_cores=NC, num_subcores=NS
)


@functools.partial(
    pl.kernel,
    out_type=jax.ShapeDtypeStruct((SEQ, NBLK, 8, 8, 128), jnp.float32),
    mesh=_MESH,
    scratch_types=[
        pltpu.VMEM((VROWS, TPAD), jnp.float32),
        pltpu.VMEM((BATCH,), jnp.int32),
        pltpu.VMEM((BATCH,), jnp.int32),
        pltpu.VMEM((VBLK, 8, 8, 128), jnp.float32),
        pltpu.VMEM((VBLK, 8, 8, 128), jnp.float32),
        pltpu.SemaphoreType.DMA,
        pltpu.SemaphoreType.DMA,
        pltpu.SemaphoreType.DMA,
        pltpu.SemaphoreType.DMA,
    ],
    compiler_params=pltpu.CompilerParams(
        use_tc_tiling_on_sc=False, needs_layout_passes=False
    ),
)
def _lookup(
    table_hbm, ids_hbm, out_hbm,
    tbl, idx0, idx1, pan0, pan1, semw0, semw1, semi0, semi1,
):
    wid = lax.axis_index("s") * NC + lax.axis_index("c")
    bs = jnp.minimum(VBLK * wid, LAST_START)

    pltpu.sync_copy(table_hbm.at[pl.ds(8 * bs, VROWS)], tbl)

    idxs = (idx0, idx1)
    pans = (pan0, pan1)
    semw = (semw0, semw1)
    semi = (semi0, semi1)

    def start_idx(s, p):
        pltpu.async_copy(ids_hbm.at[s], idxs[p], semi[p])

    def wait_idx(p):
        pltpu.make_async_copy(ids_hbm.at[0], idxs[p], semi[p]).wait()

    def compute(p):
        @plsc.parallel_loop(0, NGRP, 1, unroll=8)
        def _(g):
            bt = g // 8
            gr = g - 8 * bt
            tv = idxs[p][pl.ds(g * 16, 16)]
            for vb in range(VBLK):
                for vs in range(8):
                    val = plsc.load_gather(tbl.at[8 * vb + vs], [tv])
                    pans[p][vb, bt, vs, pl.ds(gr * 16, 16)] = val

    def start_write(s, p):
        pltpu.async_copy(pans[p], out_hbm.at[s, pl.ds(bs, VBLK)], semw[p])

    def wait_write(p):
        pltpu.make_async_copy(pans[p], out_hbm.at[0, pl.ds(bs, VBLK)], semw[p]).wait()

    start_idx(0, 0)
    start_idx(1, 1)
    wait_idx(0)
    compute(0)
    start_write(0, 0)
    wait_idx(1)
    compute(1)
    start_write(1, 1)

    def pair(t, carry):
        s0 = 2 * t
        start_idx(s0, 0)
        start_idx(s0 + 1, 1)
        wait_write(0)
        wait_idx(0)
        compute(0)
        start_write(s0, 0)
        wait_write(1)
        wait_idx(1)
        compute(1)
        start_write(s0 + 1, 1)
        return carry

    lax.fori_loop(1, SEQ // 2, pair, 0)
    wait_write(0)
    wait_write(1)


def kernel(input_ids, embed, W, b):
    table = _compute_table(embed, W, b)
    ids_t = input_ids.T.astype(jnp.int32)
    out5 = _lookup(table, ids_t)
    x = out5.transpose(2, 4, 0, 1, 3)
    return x.reshape(BATCH, SEQ, VOCAB)
